# baseline (speedup 1.0000x reference)
import functools

import jax
import jax.numpy as jnp
from jax import lax
from jax.experimental import pallas as pl
from jax.experimental.pallas import tpu as pltpu
from jax.experimental.pallas import tpu_sc as plsc

EMBED = 64
CLASSES = 1000
BATCH = 4096
SEQ = 200

NUM_CORES = 2
NUM_SUBCORES = 16
NUM_WORKERS = NUM_CORES * NUM_SUBCORES
BAGS_PER_W = BATCH // NUM_WORKERS
CHUNK = 100
CHUNKS_PER_BAG = SEQ // CHUNK
LANES = 16
COLS = EMBED // LANES

_mesh = plsc.VectorSubcoreMesh(core_axis_name="c", subcore_axis_name="s")


@functools.partial(
    pl.kernel,
    mesh=_mesh,
    out_type=jax.ShapeDtypeStruct((BATCH, EMBED), jnp.float32),
    compiler_params=pltpu.CompilerParams(use_tc_tiling_on_sc=False),
    scratch_types=[
        pltpu.VMEM((BAGS_PER_W * CHUNKS_PER_BAG, CHUNK), jnp.int32),
        pltpu.VMEM((CHUNK, EMBED), jnp.float32),
        pltpu.VMEM((CHUNK, EMBED), jnp.float32),
        pltpu.VMEM((BAGS_PER_W, EMBED), jnp.float32),
        pltpu.SemaphoreType.DMA,
        pltpu.SemaphoreType.DMA,
    ],
)
def _embed_bag(idx_hbm, table_hbm, y_hbm, idx_v, rows0, rows1, y_v, sem0, sem1):
    wid = lax.axis_index("s") * NUM_CORES + lax.axis_index("c")
    n_chunks = BAGS_PER_W * CHUNKS_PER_BAG
    pltpu.sync_copy(idx_hbm.at[pl.ds(wid * n_chunks, n_chunks)], idx_v)
    pltpu.async_copy(table_hbm.at[idx_v.at[0]], rows0, sem0)

    def _reduce(rows, accs):
        def body(r, accs):
            return tuple(
                a + rows[r, pl.ds(LANES * k, LANES)] for k, a in enumerate(accs)
            )
        return lax.fori_loop(0, CHUNK, body, accs, unroll=4)

    def bag(i, carry):
        c0 = CHUNKS_PER_BAG * i
        pltpu.async_copy(table_hbm.at[idx_v.at[c0 + 1]], rows1, sem1)
        pltpu.make_async_copy(table_hbm.at[pl.ds(0, CHUNK)], rows0, sem0).wait()
        z = jnp.zeros((LANES,), jnp.float32)
        accs = _reduce(rows0, (z,) * COLS)

        @pl.when(i < BAGS_PER_W - 1)
        def _():
            pltpu.async_copy(table_hbm.at[idx_v.at[c0 + 2]], rows0, sem0)

        pltpu.make_async_copy(table_hbm.at[pl.ds(0, CHUNK)], rows1, sem1).wait()
        accs = _reduce(rows1, accs)
        for k in range(COLS):
            y_v[i, pl.ds(LANES * k, LANES)] = accs[k]
        return 0

    lax.fori_loop(0, BAGS_PER_W, bag, 0)
    pltpu.sync_copy(y_v, y_hbm.at[pl.ds(wid * BAGS_PER_W, BAGS_PER_W)])


_BB = 1024


def _dense_body(y_ref, w_ref, b_ref, o_ref):
    o_ref[...] = (
        lax.dot_general(
            y_ref[...], w_ref[...], (((1,), (1,)), ((), ())),
            preferred_element_type=jnp.float32,
        )
        + b_ref[...]
    )


def _dense(y, w, b2):
    return pl.pallas_call(
        _dense_body,
        grid=(BATCH // _BB,),
        in_specs=[
            pl.BlockSpec((_BB, EMBED), lambda i: (i, 0)),
            pl.BlockSpec((CLASSES, EMBED), lambda i: (0, 0)),
            pl.BlockSpec((1, CLASSES), lambda i: (0, 0)),
        ],
        out_specs=pl.BlockSpec((_BB, CLASSES), lambda i: (i, 0)),
        out_shape=jax.ShapeDtypeStruct((BATCH, CLASSES), jnp.float32),
        compiler_params=pltpu.CompilerParams(
            dimension_semantics=("parallel",),
        ),
    )(y, w, b2)


def kernel(input, table, W, b):
    idx = input.reshape(BATCH * CHUNKS_PER_BAG, CHUNK)
    y = _embed_bag(idx, table)
    return _dense(y, W, b.reshape(1, CLASSES))
